# baseline (speedup 1.0000x reference)
import jax
import jax.numpy as jnp
from jax import lax
from jax.experimental import pallas as pl
from jax.experimental.pallas import tpu as pltpu
from jax.experimental.pallas import tpu_sc as plsc

_NC = 2
_NS = 16
_NW = _NC * _NS
_L = 16

_B, _S, _H = 4, 8192, 1024
_RW = _S // _NW
_CR = 8
_NCHUNK = _RW // _CR
_NXBUF = 2 * _B
_NPBUF = 2


def _sc_body(x_hbm, pe_hbm, out_hbm, scratch):
    pe_bufs = scratch["pe"]
    x_bufs = scratch["x"]
    pe_sems = scratch["pe_sem"]
    in_sems = scratch["in_sem"]
    out_sems = scratch["out_sem"]

    cid = lax.axis_index("c")
    sid = lax.axis_index("s")
    wid = sid * _NC + cid
    base = wid * _RW

    def rows(ci):
        return pl.ds(base + ci * _CR, _CR)

    def bufs(ci):
        g = (ci % 2) * _B
        return [g + b for b in range(_B)]

    descs_in = [None] * _NXBUF
    descs_out = [None] * _NXBUF
    descs_pe = [None] * _NPBUF

    descs_pe[0] = pltpu.async_copy(pe_hbm.at[rows(0)], pe_bufs[0], pe_sems[0])
    descs_pe[1] = pltpu.async_copy(pe_hbm.at[rows(1)], pe_bufs[1], pe_sems[1])
    for b, j in enumerate(bufs(0)):
        descs_in[j] = pltpu.async_copy(
            x_hbm.at[b, rows(0)], x_bufs[j], in_sems[j])

    for ci in range(_NCHUNK):
        cur = bufs(ci)
        if ci + 1 < _NCHUNK:
            for b, j in enumerate(bufs(ci + 1)):
                if descs_out[j] is not None:
                    descs_out[j].wait()
                    descs_out[j] = None
                descs_in[j] = pltpu.async_copy(
                    x_hbm.at[b, rows(ci + 1)], x_bufs[j], in_sems[j])
        descs_pe[ci % _NPBUF].wait()
        descs_pe[ci % _NPBUF] = None
        for j in cur:
            descs_in[j].wait()
            descs_in[j] = None

        pe_v = pe_bufs[ci % _NPBUF]
        xb = [x_bufs[j] for j in cur]

        @plsc.parallel_loop(0, _CR * _H // _L, unroll=16)
        def _(i):
            r = lax.shift_right_logical(i, 6)
            c = pl.multiple_of(
                lax.shift_left(lax.bitwise_and(i, 63), 4), _L)
            v = pe_v[r, pl.ds(c, _L)]
            for xv in xb:
                plsc.addupdate(xv.at[r, pl.ds(c, _L)], v)

        if ci + 2 < _NCHUNK:
            descs_pe[ci % _NPBUF] = pltpu.async_copy(
                pe_hbm.at[rows(ci + 2)], pe_bufs[ci % _NPBUF],
                pe_sems[ci % _NPBUF])

        for b, j in enumerate(cur):
            descs_out[j] = pltpu.async_copy(
                x_bufs[j], out_hbm.at[b, rows(ci)], out_sems[j])

    for j in range(_NXBUF):
        if descs_out[j] is not None:
            descs_out[j].wait()


def kernel(x, pe):
    B, S, H = x.shape
    mesh = plsc.VectorSubcoreMesh(core_axis_name="c", subcore_axis_name="s")
    return pl.kernel(
        _sc_body,
        out_type=jax.ShapeDtypeStruct((B, S, H), jnp.float32),
        mesh=mesh,
        scratch_types=[{
            "pe": [pltpu.VMEM((_CR, _H), jnp.float32)] * _NPBUF,
            "x": [pltpu.VMEM((_CR, _H), jnp.float32)] * _NXBUF,
            "pe_sem": [pltpu.SemaphoreType.DMA] * _NPBUF,
            "in_sem": [pltpu.SemaphoreType.DMA] * _NXBUF,
            "out_sem": [pltpu.SemaphoreType.DMA] * _NXBUF,
        }],
    )(x, pe)


# speedup vs baseline: 1.0267x; 1.0267x over previous
import jax
import jax.numpy as jnp
from jax import lax
from jax.experimental import pallas as pl
from jax.experimental.pallas import tpu as pltpu
from jax.experimental.pallas import tpu_sc as plsc

_NC = 2
_NS = 16
_NW = _NC * _NS
_L = 16

_B, _S, _H = 4, 8192, 1024
_RW = _S // _NW
_CR = 8
_NCHUNK = _RW // _CR
_NXBUF = 2 * _B
_NPBUF = 2


def _sc_body(x_hbm, pe_hbm, out_hbm, scratch):
    pe_bufs = scratch["pe"]
    x_bufs = scratch["x"]
    pe_sems = scratch["pe_sem"]
    in_sems = scratch["in_sem"]
    out_sems = scratch["out_sem"]

    cid = lax.axis_index("c")
    sid = lax.axis_index("s")
    wid = sid * _NC + cid
    base = wid * _RW

    def rows(ci):
        return pl.ds(base + ci * _CR, _CR)

    def bufs(ci):
        g = (ci % 2) * _B
        return [g + b for b in range(_B)]

    descs_in = [None] * _NXBUF
    descs_out = [None] * _NXBUF
    descs_pe = [None] * _NPBUF

    descs_pe[0] = pltpu.async_copy(pe_hbm.at[rows(0)], pe_bufs[0], pe_sems[0])
    descs_pe[1] = pltpu.async_copy(pe_hbm.at[rows(1)], pe_bufs[1], pe_sems[1])
    for b, j in enumerate(bufs(0)):
        descs_in[j] = pltpu.async_copy(
            x_hbm.at[b, rows(0)], x_bufs[j], in_sems[j])

    for ci in range(_NCHUNK):
        cur = bufs(ci)
        if ci + 1 < _NCHUNK:
            for b, j in enumerate(bufs(ci + 1)):
                if descs_out[j] is not None:
                    descs_out[j].wait()
                    descs_out[j] = None
                descs_in[j] = pltpu.async_copy(
                    x_hbm.at[b, rows(ci + 1)], x_bufs[j], in_sems[j])
        descs_pe[ci % _NPBUF].wait()
        descs_pe[ci % _NPBUF] = None
        for j in cur:
            descs_in[j].wait()
            descs_in[j] = None

        pe_v = pe_bufs[ci % _NPBUF]
        xb = [x_bufs[j] for j in cur]

        @plsc.parallel_loop(0, _CR * _H // _L, unroll=8)
        def _(i):
            r = lax.shift_right_logical(i, 6)
            c = pl.multiple_of(
                lax.shift_left(lax.bitwise_and(i, 63), 4), _L)
            v = pe_v[r, pl.ds(c, _L)]
            for xv in xb:
                plsc.addupdate(xv.at[r, pl.ds(c, _L)], v)

        if ci + 2 < _NCHUNK:
            descs_pe[ci % _NPBUF] = pltpu.async_copy(
                pe_hbm.at[rows(ci + 2)], pe_bufs[ci % _NPBUF],
                pe_sems[ci % _NPBUF])

        for b, j in enumerate(cur):
            descs_out[j] = pltpu.async_copy(
                x_bufs[j], out_hbm.at[b, rows(ci)], out_sems[j])

    for j in range(_NXBUF):
        if descs_out[j] is not None:
            descs_out[j].wait()


def kernel(x, pe):
    B, S, H = x.shape
    mesh = plsc.VectorSubcoreMesh(core_axis_name="c", subcore_axis_name="s")
    return pl.kernel(
        _sc_body,
        out_type=jax.ShapeDtypeStruct((B, S, H), jnp.float32),
        mesh=mesh,
        scratch_types=[{
            "pe": [pltpu.VMEM((_CR, _H), jnp.float32)] * _NPBUF,
            "x": [pltpu.VMEM((_CR, _H), jnp.float32)] * _NXBUF,
            "pe_sem": [pltpu.SemaphoreType.DMA] * _NPBUF,
            "in_sem": [pltpu.SemaphoreType.DMA] * _NXBUF,
            "out_sem": [pltpu.SemaphoreType.DMA] * _NXBUF,
        }],
    )(x, pe)
